# Initial kernel scaffold; baseline (speedup 1.0000x reference)
import jax
import jax.numpy as jnp
from jax.experimental import pallas as pl


def kernel(x, cb0, cb1):
    raise NotImplementedError("write your pallas kernel here")


# speedup vs baseline: 1.0380x; 1.0380x over previous
import jax
import jax.numpy as jnp
from jax.experimental import pallas as pl

_B, _T, _D = 16, 576, 256
_N = _B * _T
_K = 1024
_BLK = 1024
_GRID = _N // _BLK
_COMMIT = 0.25


def _rvq_body(x_ref, e0_ref, e1_ref, e20_ref, e21_ref,
              q_ref, c0_ref, c1_ref, loss_ref):
    x = x_ref[...]
    step = pl.program_id(0)

    def stage(r, e, e2):
        r2 = jnp.sum(r * r, axis=1, keepdims=True)
        re = jax.lax.dot_general(
            r, e, (((1,), (1,)), ((), ())),
            preferred_element_type=jnp.float32)
        dist = r2 + e2 - 2.0 * re
        iota = jax.lax.broadcasted_iota(jnp.int32, (_BLK, _K), 1)
        m = jnp.min(dist, axis=1, keepdims=True)
        idx = jnp.min(jnp.where(dist == m, iota, _K), axis=1).astype(jnp.int32)
        onehot = (iota == idx[:, None]).astype(jnp.float32)
        q = jax.lax.dot_general(
            onehot, e, (((1,), (0,)), ((), ())),
            precision=jax.lax.Precision.HIGHEST,
            preferred_element_type=jnp.float32)
        return q, idx

    e0 = e0_ref[...]
    e1 = e1_ref[...]
    q0, i0 = stage(x, e0, e20_ref[...])
    r1 = x - q0
    q1, i1 = stage(r1, e1, e21_ref[...])

    q_ref[...] = x + ((q0 + q1) - x)
    c0_ref[0, 0, :] = i0
    c1_ref[0, 0, :] = i1

    part = ((jnp.sum((q0 - x) ** 2) + jnp.sum((q1 - r1) ** 2))
            * ((1.0 + _COMMIT) / (_N * _D))).reshape(1, 1)

    @pl.when(step == 0)
    def _init():
        loss_ref[...] = jnp.zeros_like(loss_ref)

    loss_ref[...] += part


def kernel(x, cb0, cb1):
    b, t, d = x.shape
    xf = x.reshape(b * t, d)
    e20 = (cb0 ** 2).sum(axis=1)[None, :]
    e21 = (cb1 ** 2).sum(axis=1)[None, :]
    q, c0, c1, loss = pl.pallas_call(
        _rvq_body,
        grid=(_GRID,),
        in_specs=[
            pl.BlockSpec((_BLK, _D), lambda i: (i, 0)),
            pl.BlockSpec((_K, _D), lambda i: (0, 0)),
            pl.BlockSpec((_K, _D), lambda i: (0, 0)),
            pl.BlockSpec((1, _K), lambda i: (0, 0)),
            pl.BlockSpec((1, _K), lambda i: (0, 0)),
        ],
        out_specs=[
            pl.BlockSpec((_BLK, _D), lambda i: (i, 0)),
            pl.BlockSpec((1, 1, _BLK), lambda i: (i, 0, 0)),
            pl.BlockSpec((1, 1, _BLK), lambda i: (i, 0, 0)),
            pl.BlockSpec((1, 1), lambda i: (0, 0)),
        ],
        out_shape=[
            jax.ShapeDtypeStruct((_N, _D), jnp.float32),
            jax.ShapeDtypeStruct((_GRID, 1, _BLK), jnp.int32),
            jax.ShapeDtypeStruct((_GRID, 1, _BLK), jnp.int32),
            jax.ShapeDtypeStruct((1, 1), jnp.float32),
        ],
    )(xf, cb0, cb1, e20, e21)
    quantized = q.reshape(b, t, d)
    codes = jnp.stack([c0.reshape(b, t), c1.reshape(b, t)], axis=0)
    return quantized, codes, loss[0, 0]


# speedup vs baseline: 1.5804x; 1.5225x over previous
import jax
import jax.numpy as jnp
from jax.experimental import pallas as pl

_B, _T, _D = 16, 576, 256
_N = _B * _T
_K = 1024
_BLK = 1024
_GRID = _N // _BLK
_COMMIT = 0.25


def _rvq_body(x_ref, e0_ref, e1_ref, e20_ref, e21_ref,
              s0_ref, s1_ref, q_ref, c0_ref, c1_ref, loss_ref):
    x = x_ref[...]
    step = pl.program_id(0)

    def stage(r, e, e2, esplit):
        r2 = jnp.sum(r * r, axis=1, keepdims=True)
        re = jax.lax.dot_general(
            r, e, (((1,), (1,)), ((), ())),
            preferred_element_type=jnp.float32)
        dist = r2 + e2 - 2.0 * re
        iota = jax.lax.broadcasted_iota(jnp.int32, (_BLK, _K), 1)
        m = jnp.min(dist, axis=1, keepdims=True)
        idx = jnp.min(jnp.where(dist == m, iota, _K), axis=1).astype(jnp.int32)
        onehot = (iota == idx[:, None]).astype(jnp.bfloat16)
        parts = [
            jax.lax.dot_general(
                onehot, s, (((1,), (0,)), ((), ())),
                preferred_element_type=jnp.float32)
            for s in esplit
        ]
        q = (parts[0] + parts[1]) + parts[2]
        return q, idx

    e0 = e0_ref[...]
    e1 = e1_ref[...]
    s0 = [s0_ref[j] for j in range(3)]
    s1 = [s1_ref[j] for j in range(3)]
    q0, i0 = stage(x, e0, e20_ref[...], s0)
    r1 = x - q0
    q1, i1 = stage(r1, e1, e21_ref[...], s1)

    q_ref[...] = x + ((q0 + q1) - x)
    c0_ref[0, 0, :] = i0
    c1_ref[0, 0, :] = i1

    part = ((jnp.sum((q0 - x) ** 2) + jnp.sum((q1 - r1) ** 2))
            * ((1.0 + _COMMIT) / (_N * _D))).reshape(1, 1)

    @pl.when(step == 0)
    def _init():
        loss_ref[...] = jnp.zeros_like(loss_ref)

    loss_ref[...] += part


def kernel(x, cb0, cb1):
    b, t, d = x.shape
    xf = x.reshape(b * t, d)
    e20 = (cb0 ** 2).sum(axis=1)[None, :]
    e21 = (cb1 ** 2).sum(axis=1)[None, :]

    def split3(e):
        hi = e.astype(jnp.bfloat16)
        r1_ = e - hi.astype(jnp.float32)
        mid = r1_.astype(jnp.bfloat16)
        lo = (r1_ - mid.astype(jnp.float32)).astype(jnp.bfloat16)
        return jnp.stack([hi, mid, lo], axis=0)

    s0 = split3(cb0)
    s1 = split3(cb1)
    q, c0, c1, loss = pl.pallas_call(
        _rvq_body,
        grid=(_GRID,),
        in_specs=[
            pl.BlockSpec((_BLK, _D), lambda i: (i, 0)),
            pl.BlockSpec((_K, _D), lambda i: (0, 0)),
            pl.BlockSpec((_K, _D), lambda i: (0, 0)),
            pl.BlockSpec((1, _K), lambda i: (0, 0)),
            pl.BlockSpec((1, _K), lambda i: (0, 0)),
            pl.BlockSpec((3, _K, _D), lambda i: (0, 0, 0)),
            pl.BlockSpec((3, _K, _D), lambda i: (0, 0, 0)),
        ],
        out_specs=[
            pl.BlockSpec((_BLK, _D), lambda i: (i, 0)),
            pl.BlockSpec((1, 1, _BLK), lambda i: (i, 0, 0)),
            pl.BlockSpec((1, 1, _BLK), lambda i: (i, 0, 0)),
            pl.BlockSpec((1, 1), lambda i: (0, 0)),
        ],
        out_shape=[
            jax.ShapeDtypeStruct((_N, _D), jnp.float32),
            jax.ShapeDtypeStruct((_GRID, 1, _BLK), jnp.int32),
            jax.ShapeDtypeStruct((_GRID, 1, _BLK), jnp.int32),
            jax.ShapeDtypeStruct((1, 1), jnp.float32),
        ],
    )(xf, cb0, cb1, e20, e21, s0, s1)
    quantized = q.reshape(b, t, d)
    codes = jnp.stack([c0.reshape(b, t), c1.reshape(b, t)], axis=0)
    return quantized, codes, loss[0, 0]
